# baseline (speedup 1.0000x reference)
import functools

import jax
import jax.numpy as jnp
from jax import lax
from jax.experimental import pallas as pl
from jax.experimental.pallas import tpu as pltpu
from jax.experimental.pallas import tpu_sc as plsc

NC, NS = 2, 16
NW = NC * NS
NB = 4
PD = 128
PH = 64
L = 16


@functools.lru_cache(maxsize=None)
def _build(BATCH, HIST, D):
    b_per_w = BATCH // NW
    n_chunks = b_per_w // NB
    assert n_chunks % 2 == 0 and n_chunks >= 6

    mesh = plsc.VectorSubcoreMesh(
        core_axis_name="c", subcore_axis_name="s",
        num_cores=NC, num_subcores=NS)

    @functools.partial(
        pl.kernel,
        mesh=mesh,
        compiler_params=pltpu.CompilerParams(use_tc_tiling_on_sc=True),
        out_type=jax.ShapeDtypeStruct((BATCH, HIST, D), jnp.float32),
        scratch_types=[
            pltpu.VMEM((2 * NB, PH), jnp.int32),
            pltpu.VMEM((2 * NB, PH, PD), jnp.float32),
            pltpu.VMEM((2 * NB, HIST, D), jnp.float32),
            pltpu.SemaphoreType.DMA((2,)),
            pltpu.SemaphoreType.DMA((2,)),
        ],
    )
    def gather_kernel(idx_hbm, table_hbm, out_hbm, idx_v, rows_v, comp_v,
                      gsem, osem):
        wid = lax.axis_index("s") * NC + lax.axis_index("c")
        b_base = wid * b_per_w

        def fire(c, s):
            b0 = b_base + c * NB
            pltpu.sync_copy(idx_hbm.at[pl.ds(b0, NB)],
                            idx_v.at[pl.ds(s * NB, NB)])
            for i in range(NB):
                pltpu.async_copy(
                    table_hbm.at[idx_v.at[s * NB + i]],
                    rows_v.at[s * NB + i],
                    gsem.at[s])

        def wait_gathers(s):
            for i in range(NB):
                pltpu.make_async_copy(
                    table_hbm.at[pl.ds(0, PH)], rows_v.at[s * NB + i],
                    gsem.at[s]).wait()

        def compact(s):
            def comp_row(h, carry):
                for i in range(NB):
                    for j in range(D // L):
                        comp_v[s * NB + i, h, pl.ds(j * L, L)] = (
                            rows_v[s * NB + i, h, pl.ds(j * L, L)])
                return carry

            lax.fori_loop(0, HIST, comp_row, 0)

        def write(c, s):
            b0 = b_base + c * NB
            pltpu.async_copy(
                comp_v.at[pl.ds(s * NB, NB)], out_hbm.at[pl.ds(b0, NB)],
                osem.at[s])

        def drain_out(s):
            pltpu.make_async_copy(
                out_hbm.at[pl.ds(b_base, NB)], comp_v.at[pl.ds(s * NB, NB)],
                osem.at[s]).wait()

        def retire(c, s):
            wait_gathers(s)
            compact(s)
            write(c, s)

        fire(0, 0)
        fire(1, 1)
        retire(0, 0)
        drain_out(0)
        fire(2, 0)
        retire(1, 1)

        def body(g, carry):
            c0 = 2 * g
            drain_out(1)
            fire(c0 + 1, 1)
            retire(c0, 0)
            drain_out(0)
            fire(c0 + 2, 0)
            retire(c0 + 1, 1)
            return carry

        lax.fori_loop(1, n_chunks // 2 - 1, body, 0)

        c0 = n_chunks - 2
        drain_out(1)
        fire(c0 + 1, 1)
        retire(c0, 0)
        retire(c0 + 1, 1)
        drain_out(0)
        drain_out(1)

    return gather_kernel


def kernel(token_ids, w):
    BATCH, HIST = token_ids.shape
    V, D = w.shape
    idx2 = jnp.pad(token_ids.astype(jnp.int32), ((0, 0), (0, PH - HIST)))
    w_pad = jnp.pad(w, ((0, 0), (0, PD - D)))
    return _build(BATCH, HIST, D)(idx2, w_pad)
